# baseline (speedup 1.0000x reference)
import functools

import jax
import jax.numpy as jnp
from jax import lax
from jax.experimental import pallas as pl
from jax.experimental.pallas import tpu as pltpu
from jax.experimental.pallas import tpu_sc as plsc

_NUM_POS = 8192
_EMB = 1024
_SC_ROWS = 2048
_CHUNK = 56
_NBUF = 2
_TC_BLK = 512


@functools.cache
def _sc_copy_kernel():
    info = plsc.get_sparse_core_info()
    nc, ns = info.num_cores, info.num_subcores
    nw = nc * ns
    rows_per_w = _SC_ROWS // nw
    chunks = []
    rem = rows_per_w
    while rem > 0:
        c = min(rem, _CHUNK)
        chunks.append(c)
        rem -= c
    mesh = plsc.VectorSubcoreMesh(core_axis_name="c", subcore_axis_name="s")

    @functools.partial(
        pl.kernel,
        mesh=mesh,
        out_type=jax.ShapeDtypeStruct((_SC_ROWS, _EMB), jnp.float32),
        scratch_types=[
            pltpu.VMEM((_NBUF, _CHUNK, _EMB), jnp.float32),
            pltpu.SemaphoreType.DMA,
            pltpu.SemaphoreType.DMA,
            pltpu.SemaphoreType.DMA,
            pltpu.SemaphoreType.DMA,
        ],
    )
    def copy_k(table_hbm, out_hbm, buf, si0, si1, so0, so1):
        sin = (si0, si1)
        sout = (so0, so1)
        wid = lax.axis_index("s") * nc + lax.axis_index("c")
        base = wid * rows_per_w
        offs = [sum(chunks[:i]) for i in range(len(chunks))]
        n = len(chunks)
        hin = [None] * n
        hout = [None] * n

        def start_in(i):
            b = i % _NBUF
            if i >= _NBUF:
                hout[i - _NBUF].wait()
            hin[i] = pltpu.async_copy(
                table_hbm.at[pl.ds(base + offs[i], chunks[i])],
                buf.at[b, pl.ds(0, chunks[i])], sin[b])

        start_in(0)
        for i in range(n):
            if i + 1 < n:
                start_in(i + 1)
            b = i % _NBUF
            hin[i].wait()
            hout[i] = pltpu.async_copy(
                buf.at[b, pl.ds(0, chunks[i])],
                out_hbm.at[pl.ds(base + offs[i], chunks[i])], sout[b])
        for i in range(max(n - _NBUF, 0), n):
            hout[i].wait()

    return copy_k


@functools.cache
def _tc_copy_kernel():
    nblk = (_NUM_POS - _SC_ROWS) // _TC_BLK
    off = _SC_ROWS // _TC_BLK

    def body(w_ref, o_ref):
        o_ref[...] = w_ref[...]

    return pl.pallas_call(
        body,
        grid=(nblk,),
        in_specs=[pl.BlockSpec((_TC_BLK, _EMB), lambda i: (i + off, 0))],
        out_specs=pl.BlockSpec((_TC_BLK, _EMB), lambda i: (i + off, 0)),
        out_shape=jax.ShapeDtypeStruct((_NUM_POS, _EMB), jnp.float32),
    )


def kernel(x, pos_embed_weight):
    del x
    sc_out = _sc_copy_kernel()(pos_embed_weight)
    tc_full = _tc_copy_kernel()(pos_embed_weight)
    out = lax.dynamic_update_slice(tc_full, sc_out, (0, 0))
    return out[None]


# speedup vs baseline: 1.1357x; 1.1357x over previous
import functools

import jax
import jax.numpy as jnp
from jax import lax
from jax.experimental import pallas as pl
from jax.experimental.pallas import tpu as pltpu
from jax.experimental.pallas import tpu_sc as plsc

_NUM_POS = 8192
_EMB = 1024
_CHUNK = 56
_NBUF = 2


@functools.cache
def _copy_kernel():
    info = plsc.get_sparse_core_info()
    nc, ns = info.num_cores, info.num_subcores
    nw = nc * ns
    rows_per_w = _NUM_POS // nw
    chunks = []
    rem = rows_per_w
    while rem > 0:
        c = min(rem, _CHUNK)
        chunks.append(c)
        rem -= c
    mesh = plsc.VectorSubcoreMesh(core_axis_name="c", subcore_axis_name="s")

    @functools.partial(
        pl.kernel,
        mesh=mesh,
        out_type=jax.ShapeDtypeStruct((_NUM_POS, _EMB), jnp.float32),
        scratch_types=[
            pltpu.VMEM((_NBUF, _CHUNK, _EMB), jnp.float32),
            pltpu.SemaphoreType.DMA,
            pltpu.SemaphoreType.DMA,
            pltpu.SemaphoreType.DMA,
            pltpu.SemaphoreType.DMA,
        ],
    )
    def copy_k(table_hbm, out_hbm, buf, si0, si1, so0, so1):
        sin = (si0, si1)
        sout = (so0, so1)
        wid = lax.axis_index("s") * nc + lax.axis_index("c")
        base = wid * rows_per_w
        offs = [sum(chunks[:i]) for i in range(len(chunks))]
        n = len(chunks)
        hin = [None] * n
        hout = [None] * n

        def start_in(i):
            b = i % _NBUF
            if i >= _NBUF:
                hout[i - _NBUF].wait()
            hin[i] = pltpu.async_copy(
                table_hbm.at[pl.ds(base + offs[i], chunks[i])],
                buf.at[b, pl.ds(0, chunks[i])], sin[b])

        start_in(0)
        for i in range(n):
            if i + 1 < n:
                start_in(i + 1)
            b = i % _NBUF
            hin[i].wait()
            hout[i] = pltpu.async_copy(
                buf.at[b, pl.ds(0, chunks[i])],
                out_hbm.at[pl.ds(base + offs[i], chunks[i])], sout[b])
        for i in range(max(n - _NBUF, 0), n):
            hout[i].wait()

    return copy_k


def kernel(x, pos_embed_weight):
    del x
    out = _copy_kernel()(pos_embed_weight)
    return out[None]


# speedup vs baseline: 1.1485x; 1.0113x over previous
import functools

import jax
import jax.numpy as jnp
from jax import lax
from jax.experimental import pallas as pl
from jax.experimental.pallas import tpu as pltpu
from jax.experimental.pallas import tpu_sc as plsc

_NUM_POS = 8192
_EMB = 1024
_BUFROWS = (64, 56)
_NBUF = 2


@functools.cache
def _copy_kernel():
    info = plsc.get_sparse_core_info()
    nc, ns = info.num_cores, info.num_subcores
    nw = nc * ns
    rows_per_w = _NUM_POS // nw
    chunks = []
    rem = rows_per_w
    i = 0
    while rem > 0:
        c = min(rem, _BUFROWS[i % _NBUF])
        chunks.append(c)
        rem -= c
        i += 1
    mesh = plsc.VectorSubcoreMesh(core_axis_name="c", subcore_axis_name="s")

    @functools.partial(
        pl.kernel,
        mesh=mesh,
        out_type=jax.ShapeDtypeStruct((_NUM_POS, _EMB), jnp.float32),
        scratch_types=[
            pltpu.VMEM((_BUFROWS[0], _EMB), jnp.float32),
            pltpu.VMEM((_BUFROWS[1], _EMB), jnp.float32),
            pltpu.SemaphoreType.DMA,
            pltpu.SemaphoreType.DMA,
            pltpu.SemaphoreType.DMA,
            pltpu.SemaphoreType.DMA,
        ],
    )
    def copy_k(table_hbm, out_hbm, buf0, buf1, si0, si1, so0, so1):
        bufs = (buf0, buf1)
        sin = (si0, si1)
        sout = (so0, so1)
        wid = lax.axis_index("s") * nc + lax.axis_index("c")
        base = wid * rows_per_w
        offs = [sum(chunks[:i]) for i in range(len(chunks))]
        n = len(chunks)
        hin = [None] * n
        hout = [None] * n

        def start_in(i):
            b = i % _NBUF
            if i >= _NBUF:
                hout[i - _NBUF].wait()
            hin[i] = pltpu.async_copy(
                table_hbm.at[pl.ds(base + offs[i], chunks[i])],
                bufs[b].at[pl.ds(0, chunks[i])], sin[b])

        start_in(0)
        for i in range(n):
            if i + 1 < n:
                start_in(i + 1)
            b = i % _NBUF
            hin[i].wait()
            hout[i] = pltpu.async_copy(
                bufs[b].at[pl.ds(0, chunks[i])],
                out_hbm.at[pl.ds(base + offs[i], chunks[i])], sout[b])
        for i in range(max(n - _NBUF, 0), n):
            hout[i].wait()

    return copy_k


def kernel(x, pos_embed_weight):
    del x
    out = _copy_kernel()(pos_embed_weight)
    return out[None]
